# baseline (speedup 1.0000x reference)
import functools

import jax
import jax.numpy as jnp
from jax import lax
from jax.experimental import pallas as pl
from jax.experimental.pallas import tpu as pltpu
from jax.experimental.pallas import tpu_sc as plsc

NUM_ITEM = 1000000
EMBED_DIM = 32
BATCH = 16384

_NC = 2
_NS = 16
_NW = _NC * _NS
_BPW = BATCH // _NW
_LANES = 128
_G = 16
_DEPTH = 4
_NGRP = _BPW // _G


def _gather_body(table_t_hbm, idx_hbm, out_hbm, idx_v, blocks_v, out_v, sems):
    wid = lax.axis_index("s") * _NC + lax.axis_index("c")
    base = wid * _BPW
    pltpu.sync_copy(idx_hbm.at[pl.ds(base, _BPW)], idx_v.at[pl.ds(0, _BPW)])

    def issue(i, slot):
        q = pl.multiple_of((i >> 7) << 7, _LANES)
        pltpu.async_copy(
            table_t_hbm.at[:, pl.ds(q, _LANES)], blocks_v.at[slot],
            sems.at[slot],
        )

    d_lo = lax.iota(jnp.int32, 16)
    d_hi = d_lo + 16

    vec0 = idx_v[pl.ds(0, _G)]
    for k in range(_DEPTH):
        issue(vec0[k], k)

    def step(g, vec_cur):
        vec_next = idx_v[pl.ds((g + 1) * _G, _G)]
        for j in range(_G):
            slot = j % _DEPTH
            pltpu.make_async_copy(
                table_t_hbm.at[:, pl.ds(0, _LANES)], blocks_v.at[slot],
                sems.at[slot],
            ).wait()
            i = vec_cur[j]
            b = g * _G + j
            r = jnp.full((16,), i & 127, jnp.int32)
            s = jnp.full((16,), slot, jnp.int32)
            c = jnp.full((16,), b, jnp.int32)
            lo = plsc.load_gather(blocks_v, [s, d_lo, r])
            hi = plsc.load_gather(blocks_v, [s, d_hi, r])
            plsc.store_scatter(out_v, [d_lo, c], lo)
            plsc.store_scatter(out_v, [d_hi, c], hi)
            if j < _G - _DEPTH:
                nxt = vec_cur[j + _DEPTH]
            else:
                nxt = vec_next[j - (_G - _DEPTH)]

            @pl.when(b + _DEPTH < _BPW)
            def _():
                issue(nxt, slot)

        return vec_next

    lax.fori_loop(0, _NGRP, step, vec0)
    pltpu.sync_copy(out_v, out_hbm.at[:, pl.ds(base, _BPW)])


@jax.jit
def _lookup(table_t, idx):
    run = functools.partial(
        pl.kernel,
        out_type=jax.ShapeDtypeStruct((EMBED_DIM, BATCH), jnp.float32),
        mesh=plsc.VectorSubcoreMesh(core_axis_name="c", subcore_axis_name="s"),
        scratch_types=[
            pltpu.VMEM((_BPW + _G,), jnp.int32),
            pltpu.VMEM((_DEPTH, EMBED_DIM, _LANES), jnp.float32),
            pltpu.VMEM((EMBED_DIM, _BPW), jnp.float32),
            pltpu.SemaphoreType.DMA((_DEPTH,)),
        ],
        compiler_params=pltpu.CompilerParams(needs_layout_passes=False),
    )(_gather_body)
    return run(table_t, idx)


def kernel(item_fea, table):
    idx = item_fea[:, 0].astype(jnp.int32)
    out_t = _lookup(table.T, idx)
    return out_t.T


# speedup vs baseline: 1.2776x; 1.2776x over previous
import functools

import jax
import jax.numpy as jnp
from jax import lax
from jax.experimental import pallas as pl
from jax.experimental.pallas import tpu as pltpu
from jax.experimental.pallas import tpu_sc as plsc

NUM_ITEM = 1000000
EMBED_DIM = 32
BATCH = 16384

_NC = 2
_NS = 16
_NW = _NC * _NS
_BPW = BATCH // _NW
_LANES = 128
_G = 16
_DEPTH = 8
_NGRP = _BPW // _G


def _gather_body(table_t_hbm, idx_hbm, out_hbm, idx_v, blocks_v, out_v, sems):
    wid = lax.axis_index("s") * _NC + lax.axis_index("c")
    base = wid * _BPW
    pltpu.sync_copy(idx_hbm.at[pl.ds(base, _BPW)], idx_v.at[pl.ds(0, _BPW)])

    def issue(i, slot):
        q = pl.multiple_of((i >> 7) << 7, _LANES)
        pltpu.async_copy(
            table_t_hbm.at[:, pl.ds(q, _LANES)], blocks_v.at[slot],
            sems.at[slot],
        )

    d_lo = lax.iota(jnp.int32, 16)
    d_hi = d_lo + 16

    vec0 = idx_v[pl.ds(0, _G)]
    for k in range(_DEPTH):
        issue(vec0[k], k)

    def step(g, vec_cur):
        vec_next = idx_v[pl.ds((g + 1) * _G, _G)]
        for j in range(_G):
            slot = j % _DEPTH
            pltpu.make_async_copy(
                table_t_hbm.at[:, pl.ds(0, _LANES)], blocks_v.at[slot],
                sems.at[slot],
            ).wait()
            i = vec_cur[j]
            b = g * _G + j
            r = jnp.full((16,), i & 127, jnp.int32)
            s = jnp.full((16,), slot, jnp.int32)
            c = jnp.full((16,), b, jnp.int32)
            lo = plsc.load_gather(blocks_v, [s, d_lo, r])
            hi = plsc.load_gather(blocks_v, [s, d_hi, r])
            plsc.store_scatter(out_v, [d_lo, c], lo)
            plsc.store_scatter(out_v, [d_hi, c], hi)
            if j < _G - _DEPTH:
                nxt = vec_cur[j + _DEPTH]
            else:
                nxt = vec_next[j - (_G - _DEPTH)]

            @pl.when(b + _DEPTH < _BPW)
            def _():
                issue(nxt, slot)

        return vec_next

    lax.fori_loop(0, _NGRP, step, vec0)
    pltpu.sync_copy(out_v, out_hbm.at[:, pl.ds(base, _BPW)])


@jax.jit
def _lookup(table_t, idx):
    run = functools.partial(
        pl.kernel,
        out_type=jax.ShapeDtypeStruct((EMBED_DIM, BATCH), jnp.float32),
        mesh=plsc.VectorSubcoreMesh(core_axis_name="c", subcore_axis_name="s"),
        scratch_types=[
            pltpu.VMEM((_BPW + _G,), jnp.int32),
            pltpu.VMEM((_DEPTH, EMBED_DIM, _LANES), jnp.float32),
            pltpu.VMEM((EMBED_DIM, _BPW), jnp.float32),
            pltpu.SemaphoreType.DMA((_DEPTH,)),
        ],
        compiler_params=pltpu.CompilerParams(
            needs_layout_passes=False, skip_device_barrier=True,
        ),
    )(_gather_body)
    return run(table_t, idx)


def kernel(item_fea, table):
    idx = item_fea[:, 0].astype(jnp.int32)
    out_t = _lookup(table.T, idx)
    return out_t.T
